# Initial kernel scaffold; baseline (speedup 1.0000x reference)
import jax
import jax.numpy as jnp
from jax.experimental import pallas as pl


def kernel(x, edge_index, sec_order_edge_index, W1, a1_src, a1_dst, b1, W2, a2_i, a2_j, a2_k, b2):
    raise NotImplementedError("write your pallas kernel here")


# speedup vs baseline: 60.3158x; 60.3158x over previous
import functools

import jax
import jax.numpy as jnp
from jax import lax
from jax.experimental import pallas as pl
from jax.experimental.pallas import tpu as pltpu
from jax.experimental.pallas import tpu_sc as plsc

N = 10000
E = 160000
M = 320000
H1, C1 = 4, 16
H2, C2 = 4, 8
F1 = H1 * C1
F2 = H2 * C2
A1W = F1 + 16
A2W = F2 + 16
DW = 16
RB = 1000
NSC = 2
NTILE = 32
EPT = E // NTILE
PPT = M // NTILE
CH = 128
E_CHUNKS, E_TAIL = EPT // CH, EPT % CH
P_CHUNKS, P_TAIL = PPT // CH, PPT % CH
ZROWS = 624
ZTAIL = N - 16 * ZROWS


def _mm_body(x_ref, w_ref, o_ref):
    o_ref[...] = jnp.dot(x_ref[...], w_ref[...], preferred_element_type=jnp.float32)


def _matmul(x, w):
    n, k = x.shape
    ko, c = w.shape
    return pl.pallas_call(
        _mm_body,
        grid=(n // RB,),
        in_specs=[
            pl.BlockSpec((RB, k), lambda i: (i, 0)),
            pl.BlockSpec((k, c), lambda i: (0, 0)),
        ],
        out_specs=pl.BlockSpec((RB, c), lambda i: (i, 0)),
        out_shape=jax.ShapeDtypeStruct((n, c), jnp.float32),
    )(x, w)


def _combine1_body(p0_ref, p1_ref, em_ref, b1_ref, w2_ref, o_ref):
    t = p0_ref[...] + p1_ref[...]
    den = jnp.dot(t, em_ref[...], preferred_element_type=jnp.float32)
    num = t[:, :F1]
    out1 = jax.nn.relu(num / (den + 1e-16) + b1_ref[...])
    o_ref[...] = jnp.dot(out1, w2_ref[...], preferred_element_type=jnp.float32)


def _combine1(p0, p1, emat, b1r, we2):
    c = we2.shape[1]
    return pl.pallas_call(
        _combine1_body,
        grid=(N // RB,),
        in_specs=[
            pl.BlockSpec((RB, A1W), lambda i: (i, 0)),
            pl.BlockSpec((RB, A1W), lambda i: (i, 0)),
            pl.BlockSpec((A1W, F1), lambda i: (0, 0)),
            pl.BlockSpec((1, F1), lambda i: (0, 0)),
            pl.BlockSpec((F1, c), lambda i: (0, 0)),
        ],
        out_specs=pl.BlockSpec((RB, c), lambda i: (i, 0)),
        out_shape=jax.ShapeDtypeStruct((N, c), jnp.float32),
    )(p0, p1, emat, b1r, we2)


def _combine2_body(p0_ref, p1_ref, em_ref, b2_ref, o_ref):
    t = p0_ref[...] + p1_ref[...]
    den = jnp.dot(t, em_ref[...], preferred_element_type=jnp.float32)
    num = t[:, :F2]
    o_ref[...] = num / (den + 1e-16) + b2_ref[...]


def _combine2(p0, p1, emat, b2r):
    return pl.pallas_call(
        _combine2_body,
        grid=(N // RB,),
        in_specs=[
            pl.BlockSpec((RB, A2W), lambda i: (i, 0)),
            pl.BlockSpec((RB, A2W), lambda i: (i, 0)),
            pl.BlockSpec((A2W, F2), lambda i: (0, 0)),
            pl.BlockSpec((1, F2), lambda i: (0, 0)),
        ],
        out_specs=pl.BlockSpec((RB, F2), lambda i: (i, 0)),
        out_shape=jax.ShapeDtypeStruct((N, F2), jnp.float32),
    )(p0, p1, emat, b2r)


def _splat(vec, idx16):
    return lax.gather(
        vec, idx16.reshape(16, 1),
        lax.GatherDimensionNumbers(
            offset_dims=(), collapsed_slice_dims=(0,), start_index_map=(0,)),
        (1,), mode=lax.GatherScatterMode.PROMISE_IN_BOUNDS)


def _edge_math1(ra, rb, e):
    lanes = lax.iota(jnp.int32, 16)
    mask4 = lanes < 4
    v = ra[e, pl.ds(F1, 16)] + rb[e, :]
    v = jnp.maximum(v, 0.2 * v)
    ex = jnp.where(mask4, jnp.exp(v), 0.0)
    ra[e, pl.ds(F1, 16)] = ex
    for j in range(H1):
        m = _splat(ex, lanes * 0 + j)
        ra[e, pl.ds(C1 * j, 16)] = ra[e, pl.ds(C1 * j, 16)] * m


def _edge_math2(ra, rbj, rbk, e):
    lanes = lax.iota(jnp.int32, 16)
    mask4 = lanes < 4
    v = ra[e, pl.ds(F2, 16)] + rbj[e, :] + rbk[e, :]
    v = jnp.maximum(v, 0.2 * v)
    ex = jnp.where(mask4, jnp.exp(v), 0.0)
    ra[e, pl.ds(F2, 16)] = ex
    half = lanes >> 3
    m0 = _splat(ex, half)
    ra[e, pl.ds(0, 16)] = ra[e, pl.ds(0, 16)] * m0
    m1 = _splat(ex, half + 2)
    ra[e, pl.ds(16, 16)] = ra[e, pl.ds(16, 16)] * m1


@functools.lru_cache(maxsize=None)
def _make_gat_scatter():
    return functools.partial(
        pl.kernel,
        out_type=[jax.ShapeDtypeStruct((N, A1W), jnp.float32),
                  jax.ShapeDtypeStruct((N, A1W), jnp.float32)],
        mesh=plsc.VectorSubcoreMesh(core_axis_name="c", subcore_axis_name="s"),
        compiler_params=pltpu.CompilerParams(use_tc_tiling_on_sc=False),
        scratch_types=[
        pltpu.VMEM_SHARED((N, A1W), jnp.float32),
        pltpu.VMEM((CH, A1W), jnp.float32),
        pltpu.VMEM((CH, DW), jnp.float32),
        pltpu.VMEM((CH,), jnp.int32),
        pltpu.VMEM((CH,), jnp.int32),
        pltpu.VMEM((E_TAIL,), jnp.int32),
        pltpu.VMEM((E_TAIL,), jnp.int32),
        pltpu.SemaphoreType.DMA,
        pltpu.SemaphoreType.DMA,
        ],
    )(_gat_scatter_body)


def _gat_scatter_body(a1_hbm, d1_hbm, src_hbm, dst_hbm, z_hbm, p0_hbm, p1_hbm,
                      acc, ra, rb, idx_s, idx_d, idx_st, idx_dt, sem1, sem2):
    cid = lax.axis_index("c")
    sid = lax.axis_index("s")
    w = sid * NSC + cid

    pltpu.sync_copy(z_hbm.at[pl.ds(sid * ZROWS, ZROWS)],
                    acc.at[pl.ds(sid * ZROWS, ZROWS)])

    @pl.when(sid == 0)
    def _():
        pltpu.sync_copy(z_hbm.at[pl.ds(16 * ZROWS, ZTAIL)],
                        acc.at[pl.ds(16 * ZROWS, ZTAIL)])

    plsc.subcore_barrier()

    def chunk(c, _):
        base = w * EPT + c * CH
        pltpu.sync_copy(src_hbm.at[pl.ds(base, CH)], idx_s)
        pltpu.sync_copy(dst_hbm.at[pl.ds(base, CH)], idx_d)
        cp1 = pltpu.async_copy(a1_hbm.at[idx_s], ra, sem1)
        cp2 = pltpu.async_copy(d1_hbm.at[idx_d], rb, sem2)
        cp1.wait()
        cp2.wait()

        def edge(e, carry):
            _edge_math1(ra, rb, e)
            return carry

        lax.fori_loop(0, CH, edge, 0)
        pltpu.sync_copy(ra, acc.at[idx_d], add=True)
        return _

    lax.fori_loop(0, E_CHUNKS, chunk, 0)

    base = w * EPT + E_CHUNKS * CH
    pltpu.sync_copy(src_hbm.at[pl.ds(base, E_TAIL)], idx_st)
    pltpu.sync_copy(dst_hbm.at[pl.ds(base, E_TAIL)], idx_dt)
    cp1 = pltpu.async_copy(a1_hbm.at[idx_st], ra.at[pl.ds(0, E_TAIL)], sem1)
    cp2 = pltpu.async_copy(d1_hbm.at[idx_dt], rb.at[pl.ds(0, E_TAIL)], sem2)
    cp1.wait()
    cp2.wait()
    for e in range(E_TAIL):
        _edge_math1(ra, rb, e)
    pltpu.sync_copy(ra.at[pl.ds(0, E_TAIL)], acc.at[idx_dt], add=True)

    plsc.subcore_barrier()

    @pl.when(jnp.logical_and(sid == 0, cid == 0))
    def _():
        pltpu.sync_copy(acc, p0_hbm)

    @pl.when(jnp.logical_and(sid == 0, cid == 1))
    def _():
        pltpu.sync_copy(acc, p1_hbm)


@functools.lru_cache(maxsize=None)
def _make_path_scatter():
    return functools.partial(
        pl.kernel,
        out_type=[jax.ShapeDtypeStruct((N, A2W), jnp.float32),
                  jax.ShapeDtypeStruct((N, A2W), jnp.float32)],
        mesh=plsc.VectorSubcoreMesh(core_axis_name="c", subcore_axis_name="s"),
        compiler_params=pltpu.CompilerParams(use_tc_tiling_on_sc=False),
        scratch_types=[
        pltpu.VMEM_SHARED((N, A2W), jnp.float32),
        pltpu.VMEM((CH, A2W), jnp.float32),
        pltpu.VMEM((CH, DW), jnp.float32),
        pltpu.VMEM((CH, DW), jnp.float32),
        pltpu.VMEM((CH,), jnp.int32),
        pltpu.VMEM((CH,), jnp.int32),
        pltpu.VMEM((CH,), jnp.int32),
        pltpu.VMEM((P_TAIL,), jnp.int32),
        pltpu.VMEM((P_TAIL,), jnp.int32),
        pltpu.VMEM((P_TAIL,), jnp.int32),
        pltpu.SemaphoreType.DMA,
        pltpu.SemaphoreType.DMA,
        pltpu.SemaphoreType.DMA,
        ],
    )(_path_scatter_body)


def _path_scatter_body(a2_hbm, dj_hbm, dk_hbm, pi_hbm, pj_hbm, pk_hbm, z_hbm,
                       q0_hbm, q1_hbm, acc, ra, rbj, rbk,
                       idx_i, idx_j, idx_k, idx_it, idx_jt, idx_kt,
                       sem1, sem2, sem3):
    cid = lax.axis_index("c")
    sid = lax.axis_index("s")
    w = sid * NSC + cid

    pltpu.sync_copy(z_hbm.at[pl.ds(sid * ZROWS, ZROWS)],
                    acc.at[pl.ds(sid * ZROWS, ZROWS)])

    @pl.when(sid == 0)
    def _():
        pltpu.sync_copy(z_hbm.at[pl.ds(16 * ZROWS, ZTAIL)],
                        acc.at[pl.ds(16 * ZROWS, ZTAIL)])

    plsc.subcore_barrier()

    def chunk(c, _):
        base = w * PPT + c * CH
        pltpu.sync_copy(pi_hbm.at[pl.ds(base, CH)], idx_i)
        pltpu.sync_copy(pj_hbm.at[pl.ds(base, CH)], idx_j)
        pltpu.sync_copy(pk_hbm.at[pl.ds(base, CH)], idx_k)
        cp1 = pltpu.async_copy(a2_hbm.at[idx_i], ra, sem1)
        cp2 = pltpu.async_copy(dj_hbm.at[idx_j], rbj, sem2)
        cp3 = pltpu.async_copy(dk_hbm.at[idx_k], rbk, sem3)
        cp1.wait()
        cp2.wait()
        cp3.wait()

        def path(e, carry):
            _edge_math2(ra, rbj, rbk, e)
            return carry

        lax.fori_loop(0, CH, path, 0)
        pltpu.sync_copy(ra, acc.at[idx_k], add=True)
        return _

    lax.fori_loop(0, P_CHUNKS, chunk, 0)

    base = w * PPT + P_CHUNKS * CH
    pltpu.sync_copy(pi_hbm.at[pl.ds(base, P_TAIL)], idx_it)
    pltpu.sync_copy(pj_hbm.at[pl.ds(base, P_TAIL)], idx_jt)
    pltpu.sync_copy(pk_hbm.at[pl.ds(base, P_TAIL)], idx_kt)
    cp1 = pltpu.async_copy(a2_hbm.at[idx_it], ra.at[pl.ds(0, P_TAIL)], sem1)
    cp2 = pltpu.async_copy(dj_hbm.at[idx_jt], rbj.at[pl.ds(0, P_TAIL)], sem2)
    cp3 = pltpu.async_copy(dk_hbm.at[idx_kt], rbk.at[pl.ds(0, P_TAIL)], sem3)
    cp1.wait()
    cp2.wait()
    cp3.wait()
    for e in range(P_TAIL):
        _edge_math2(ra, rbj, rbk, e)
    pltpu.sync_copy(ra.at[pl.ds(0, P_TAIL)], acc.at[idx_kt], add=True)

    plsc.subcore_barrier()

    @pl.when(jnp.logical_and(sid == 0, cid == 0))
    def _():
        pltpu.sync_copy(acc, q0_hbm)

    @pl.when(jnp.logical_and(sid == 0, cid == 1))
    def _():
        pltpu.sync_copy(acc, q1_hbm)


def _pack_weights1(W1, a1_src, a1_dst):
    w3 = W1.reshape(300, H1, C1)
    ws = jnp.einsum('khc,hc->kh', w3, a1_src)
    wd = jnp.einsum('khc,hc->kh', w3, a1_dst)
    z = jnp.zeros((300, 12), jnp.float32)
    return jnp.concatenate([W1, ws, z, wd, z], axis=1)


def _pack_weights2(W2, a2_i, a2_j, a2_k):
    w3 = W2.reshape(F1, H2, C2)
    wi = jnp.einsum('khc,hc->kh', w3, a2_i)
    wj = jnp.einsum('khc,hc->kh', w3, a2_j)
    wk = jnp.einsum('khc,hc->kh', w3, a2_k)
    z = jnp.zeros((F1, 12), jnp.float32)
    return jnp.concatenate([W2, wi, z, wj, z, wk, z], axis=1)


def _den_expand(total_w, heads, width):
    em = jnp.kron(jnp.eye(heads, dtype=jnp.float32),
                  jnp.ones((1, width), jnp.float32))
    top = jnp.zeros((heads * width, heads * width), jnp.float32)
    bot = jnp.zeros((total_w - heads * width - heads, heads * width), jnp.float32)
    return jnp.concatenate([top, em, bot], axis=0)


def kernel(x, edge_index, sec_order_edge_index, W1, a1_src, a1_dst, b1,
           W2, a2_i, a2_j, a2_k, b2):
    src, dst = edge_index[0], edge_index[1]
    pi, pj, pk = (sec_order_edge_index[0], sec_order_edge_index[1],
                  sec_order_edge_index[2])

    we1 = _pack_weights1(W1, a1_src, a1_dst)
    y1 = _matmul(x, we1)
    a1 = y1[:, :A1W]
    d1 = y1[:, A1W:]

    z1 = jnp.zeros((N, A1W), jnp.float32)
    p0, p1 = _make_gat_scatter()(a1, d1, src, dst, z1)

    we2 = _pack_weights2(W2, a2_i, a2_j, a2_k)
    em1 = _den_expand(A1W, H1, C1)
    y2 = _combine1(p0, p1, em1, b1.reshape(1, F1), we2)
    a2 = y2[:, :A2W]
    d2j = y2[:, A2W:A2W + DW]
    d2k = y2[:, A2W + DW:]

    z2 = jnp.zeros((N, A2W), jnp.float32)
    q0, q1 = _make_path_scatter()(a2, d2j, d2k, pi, pj, pk, z2)

    em2 = _den_expand(A2W, H2, C2)
    return _combine2(q0, q1, em2, b2.reshape(1, F2))


# speedup vs baseline: 74.4209x; 1.2339x over previous
import functools

import jax
import jax.numpy as jnp
from jax import lax
from jax.experimental import pallas as pl
from jax.experimental.pallas import tpu as pltpu
from jax.experimental.pallas import tpu_sc as plsc

N = 10000
E = 160000
M = 320000
H1, C1 = 4, 16
H2, C2 = 4, 8
F1 = H1 * C1
F2 = H2 * C2
A1W = F1 + 16
A2W = F2 + 16
DW = 16
RB = 1000
NSC = 2
NTILE = 32
EPT = E // NTILE
PPT = M // NTILE
CH = 128
E_CHUNKS, E_TAIL = EPT // CH, EPT % CH
P_CHUNKS, P_TAIL = PPT // CH, PPT % CH
ZROWS = 624
ZTAIL = N - 16 * ZROWS


def _mm_body(x_ref, w_ref, o_ref):
    o_ref[...] = jnp.dot(x_ref[...], w_ref[...], preferred_element_type=jnp.float32)


def _matmul(x, w):
    n, k = x.shape
    ko, c = w.shape
    return pl.pallas_call(
        _mm_body,
        grid=(n // RB,),
        in_specs=[
            pl.BlockSpec((RB, k), lambda i: (i, 0)),
            pl.BlockSpec((k, c), lambda i: (0, 0)),
        ],
        out_specs=pl.BlockSpec((RB, c), lambda i: (i, 0)),
        out_shape=jax.ShapeDtypeStruct((n, c), jnp.float32),
    )(x, w)


def _combine1_body(p0_ref, p1_ref, em_ref, b1_ref, w2_ref, o_ref):
    t = p0_ref[...] + p1_ref[...]
    den = jnp.dot(t, em_ref[...], preferred_element_type=jnp.float32)
    num = t[:, :F1]
    out1 = jax.nn.relu(num / (den + 1e-16) + b1_ref[...])
    o_ref[...] = jnp.dot(out1, w2_ref[...], preferred_element_type=jnp.float32)


def _combine1(p0, p1, emat, b1r, we2):
    c = we2.shape[1]
    return pl.pallas_call(
        _combine1_body,
        grid=(N // RB,),
        in_specs=[
            pl.BlockSpec((RB, A1W), lambda i: (i, 0)),
            pl.BlockSpec((RB, A1W), lambda i: (i, 0)),
            pl.BlockSpec((A1W, F1), lambda i: (0, 0)),
            pl.BlockSpec((1, F1), lambda i: (0, 0)),
            pl.BlockSpec((F1, c), lambda i: (0, 0)),
        ],
        out_specs=pl.BlockSpec((RB, c), lambda i: (i, 0)),
        out_shape=jax.ShapeDtypeStruct((N, c), jnp.float32),
    )(p0, p1, emat, b1r, we2)


def _combine2_body(p0_ref, p1_ref, em_ref, b2_ref, o_ref):
    t = p0_ref[...] + p1_ref[...]
    den = jnp.dot(t, em_ref[...], preferred_element_type=jnp.float32)
    num = t[:, :F2]
    o_ref[...] = num / (den + 1e-16) + b2_ref[...]


def _combine2(p0, p1, emat, b2r):
    return pl.pallas_call(
        _combine2_body,
        grid=(N // RB,),
        in_specs=[
            pl.BlockSpec((RB, A2W), lambda i: (i, 0)),
            pl.BlockSpec((RB, A2W), lambda i: (i, 0)),
            pl.BlockSpec((A2W, F2), lambda i: (0, 0)),
            pl.BlockSpec((1, F2), lambda i: (0, 0)),
        ],
        out_specs=pl.BlockSpec((RB, F2), lambda i: (i, 0)),
        out_shape=jax.ShapeDtypeStruct((N, F2), jnp.float32),
    )(p0, p1, emat, b2r)


def _splat(vec, idx16):
    return lax.gather(
        vec, idx16.reshape(16, 1),
        lax.GatherDimensionNumbers(
            offset_dims=(), collapsed_slice_dims=(0,), start_index_map=(0,)),
        (1,), mode=lax.GatherScatterMode.PROMISE_IN_BOUNDS)


def _edge_math1(ra, rb, e):
    lanes = lax.iota(jnp.int32, 16)
    mask4 = lanes < 4
    v = ra[e, pl.ds(F1, 16)] + rb[e, :]
    v = jnp.maximum(v, 0.2 * v)
    ex = jnp.where(mask4, jnp.exp(v), 0.0)
    ra[e, pl.ds(F1, 16)] = ex
    for j in range(H1):
        m = _splat(ex, lanes * 0 + j)
        ra[e, pl.ds(C1 * j, 16)] = ra[e, pl.ds(C1 * j, 16)] * m


def _edge_math2(ra, rbj, rbk, e):
    lanes = lax.iota(jnp.int32, 16)
    mask4 = lanes < 4
    v = ra[e, pl.ds(F2, 16)] + rbj[e, :] + rbk[e, :]
    v = jnp.maximum(v, 0.2 * v)
    ex = jnp.where(mask4, jnp.exp(v), 0.0)
    ra[e, pl.ds(F2, 16)] = ex
    half = lanes >> 3
    m0 = _splat(ex, half)
    ra[e, pl.ds(0, 16)] = ra[e, pl.ds(0, 16)] * m0
    m1 = _splat(ex, half + 2)
    ra[e, pl.ds(16, 16)] = ra[e, pl.ds(16, 16)] * m1


@functools.lru_cache(maxsize=None)
def _make_gat_scatter():
    return functools.partial(
        pl.kernel,
        out_type=[jax.ShapeDtypeStruct((N, A1W), jnp.float32),
                  jax.ShapeDtypeStruct((N, A1W), jnp.float32)],
        mesh=plsc.VectorSubcoreMesh(core_axis_name="c", subcore_axis_name="s"),
        compiler_params=pltpu.CompilerParams(use_tc_tiling_on_sc=False),
        scratch_types=[
        pltpu.VMEM_SHARED((N, A1W), jnp.float32),
        pltpu.VMEM((CH, A1W), jnp.float32),
        pltpu.VMEM((CH, A1W), jnp.float32),
        pltpu.VMEM((CH, DW), jnp.float32),
        pltpu.VMEM((CH, DW), jnp.float32),
        pltpu.VMEM((CH,), jnp.int32),
        pltpu.VMEM((CH,), jnp.int32),
        pltpu.VMEM((CH,), jnp.int32),
        pltpu.VMEM((CH,), jnp.int32),
        pltpu.VMEM((E_TAIL,), jnp.int32),
        pltpu.VMEM((E_TAIL,), jnp.int32),
        pltpu.SemaphoreType.DMA,
        pltpu.SemaphoreType.DMA,
        pltpu.SemaphoreType.DMA,
        pltpu.SemaphoreType.DMA,
        ],
    )(_gat_scatter_body)


def _gat_scatter_body(a1_hbm, d1_hbm, src_hbm, dst_hbm, z_hbm, p0_hbm, p1_hbm,
                      acc, ra0, ra1, rb0, rb1, idx_s0, idx_s1, idx_d0, idx_d1,
                      idx_st, idx_dt, semg0, semg1, sems0, sems1):
    cid = lax.axis_index("c")
    sid = lax.axis_index("s")
    w = sid * NSC + cid

    pltpu.sync_copy(z_hbm.at[pl.ds(sid * ZROWS, ZROWS)],
                    acc.at[pl.ds(sid * ZROWS, ZROWS)])

    @pl.when(sid == 0)
    def _():
        pltpu.sync_copy(z_hbm.at[pl.ds(16 * ZROWS, ZTAIL)],
                        acc.at[pl.ds(16 * ZROWS, ZTAIL)])

    plsc.subcore_barrier()

    def edges(ra, rb, n):
        def edge(e, carry):
            _edge_math1(ra, rb, e)
            return carry
        lax.fori_loop(0, n, edge, 0)

    def pair(p, _):
        base_e = w * EPT + (2 * p) * CH
        base_o = base_e + CH
        pltpu.sync_copy(src_hbm.at[pl.ds(base_e, CH)], idx_s0)
        pltpu.sync_copy(dst_hbm.at[pl.ds(base_e, CH)], idx_d0)
        g0a = pltpu.async_copy(a1_hbm.at[idx_s0], ra0, semg0)
        g0b = pltpu.async_copy(d1_hbm.at[idx_d0], rb0, semg0)
        pltpu.sync_copy(src_hbm.at[pl.ds(base_o, CH)], idx_s1)
        pltpu.sync_copy(dst_hbm.at[pl.ds(base_o, CH)], idx_d1)
        g1a = pltpu.async_copy(a1_hbm.at[idx_s1], ra1, semg1)
        g1b = pltpu.async_copy(d1_hbm.at[idx_d1], rb1, semg1)
        g0a.wait()
        g0b.wait()
        edges(ra0, rb0, CH)
        s0 = pltpu.async_copy(ra0, acc.at[idx_d0], sems0, add=True)
        g1a.wait()
        g1b.wait()
        edges(ra1, rb1, CH)
        s1 = pltpu.async_copy(ra1, acc.at[idx_d1], sems1, add=True)
        s0.wait()
        s1.wait()
        return _

    lax.fori_loop(0, E_CHUNKS // 2, pair, 0)

    def chunk(c, _):
        base = w * EPT + c * CH
        pltpu.sync_copy(src_hbm.at[pl.ds(base, CH)], idx_s0)
        pltpu.sync_copy(dst_hbm.at[pl.ds(base, CH)], idx_d0)
        cp1 = pltpu.async_copy(a1_hbm.at[idx_s0], ra0, semg0)
        cp2 = pltpu.async_copy(d1_hbm.at[idx_d0], rb0, semg0)
        cp1.wait()
        cp2.wait()
        edges(ra0, rb0, CH)
        pltpu.sync_copy(ra0, acc.at[idx_d0], add=True)
        return _

    lax.fori_loop(2 * (E_CHUNKS // 2), E_CHUNKS, chunk, 0)

    base = w * EPT + E_CHUNKS * CH
    pltpu.sync_copy(src_hbm.at[pl.ds(base, E_TAIL)], idx_st)
    pltpu.sync_copy(dst_hbm.at[pl.ds(base, E_TAIL)], idx_dt)
    cp1 = pltpu.async_copy(a1_hbm.at[idx_st], ra0.at[pl.ds(0, E_TAIL)], semg0)
    cp2 = pltpu.async_copy(d1_hbm.at[idx_dt], rb0.at[pl.ds(0, E_TAIL)], semg0)
    cp1.wait()
    cp2.wait()
    for e in range(E_TAIL):
        _edge_math1(ra0, rb0, e)
    pltpu.sync_copy(ra0.at[pl.ds(0, E_TAIL)], acc.at[idx_dt], add=True)

    plsc.subcore_barrier()

    @pl.when(jnp.logical_and(sid == 0, cid == 0))
    def _():
        pltpu.sync_copy(acc, p0_hbm)

    @pl.when(jnp.logical_and(sid == 0, cid == 1))
    def _():
        pltpu.sync_copy(acc, p1_hbm)


@functools.lru_cache(maxsize=None)
def _make_path_scatter():
    return functools.partial(
        pl.kernel,
        out_type=[jax.ShapeDtypeStruct((N, A2W), jnp.float32),
                  jax.ShapeDtypeStruct((N, A2W), jnp.float32)],
        mesh=plsc.VectorSubcoreMesh(core_axis_name="c", subcore_axis_name="s"),
        compiler_params=pltpu.CompilerParams(use_tc_tiling_on_sc=False),
        scratch_types=[
        pltpu.VMEM_SHARED((N, A2W), jnp.float32),
        pltpu.VMEM((CH, A2W), jnp.float32),
        pltpu.VMEM((CH, A2W), jnp.float32),
        pltpu.VMEM((CH, DW), jnp.float32),
        pltpu.VMEM((CH, DW), jnp.float32),
        pltpu.VMEM((CH, DW), jnp.float32),
        pltpu.VMEM((CH, DW), jnp.float32),
        pltpu.VMEM((CH,), jnp.int32),
        pltpu.VMEM((CH,), jnp.int32),
        pltpu.VMEM((CH,), jnp.int32),
        pltpu.VMEM((CH,), jnp.int32),
        pltpu.VMEM((CH,), jnp.int32),
        pltpu.VMEM((CH,), jnp.int32),
        pltpu.VMEM((P_TAIL,), jnp.int32),
        pltpu.VMEM((P_TAIL,), jnp.int32),
        pltpu.VMEM((P_TAIL,), jnp.int32),
        pltpu.SemaphoreType.DMA,
        pltpu.SemaphoreType.DMA,
        pltpu.SemaphoreType.DMA,
        pltpu.SemaphoreType.DMA,
        ],
    )(_path_scatter_body)


def _path_scatter_body(a2_hbm, dj_hbm, dk_hbm, pi_hbm, pj_hbm, pk_hbm, z_hbm,
                       q0_hbm, q1_hbm, acc, ra0, ra1, rbj0, rbj1, rbk0, rbk1,
                       idx_i0, idx_i1, idx_j0, idx_j1, idx_k0, idx_k1,
                       idx_it, idx_jt, idx_kt, semg0, semg1, sems0, sems1):
    cid = lax.axis_index("c")
    sid = lax.axis_index("s")
    w = sid * NSC + cid

    pltpu.sync_copy(z_hbm.at[pl.ds(sid * ZROWS, ZROWS)],
                    acc.at[pl.ds(sid * ZROWS, ZROWS)])

    @pl.when(sid == 0)
    def _():
        pltpu.sync_copy(z_hbm.at[pl.ds(16 * ZROWS, ZTAIL)],
                        acc.at[pl.ds(16 * ZROWS, ZTAIL)])

    plsc.subcore_barrier()

    def paths(ra, rbj, rbk, n):
        def path(e, carry):
            _edge_math2(ra, rbj, rbk, e)
            return carry
        lax.fori_loop(0, n, path, 0)

    def pair(p, _):
        base_e = w * PPT + (2 * p) * CH
        base_o = base_e + CH
        pltpu.sync_copy(pi_hbm.at[pl.ds(base_e, CH)], idx_i0)
        pltpu.sync_copy(pj_hbm.at[pl.ds(base_e, CH)], idx_j0)
        pltpu.sync_copy(pk_hbm.at[pl.ds(base_e, CH)], idx_k0)
        g0a = pltpu.async_copy(a2_hbm.at[idx_i0], ra0, semg0)
        g0b = pltpu.async_copy(dj_hbm.at[idx_j0], rbj0, semg0)
        g0c = pltpu.async_copy(dk_hbm.at[idx_k0], rbk0, semg0)
        pltpu.sync_copy(pi_hbm.at[pl.ds(base_o, CH)], idx_i1)
        pltpu.sync_copy(pj_hbm.at[pl.ds(base_o, CH)], idx_j1)
        pltpu.sync_copy(pk_hbm.at[pl.ds(base_o, CH)], idx_k1)
        g1a = pltpu.async_copy(a2_hbm.at[idx_i1], ra1, semg1)
        g1b = pltpu.async_copy(dj_hbm.at[idx_j1], rbj1, semg1)
        g1c = pltpu.async_copy(dk_hbm.at[idx_k1], rbk1, semg1)
        g0a.wait()
        g0b.wait()
        g0c.wait()
        paths(ra0, rbj0, rbk0, CH)
        s0 = pltpu.async_copy(ra0, acc.at[idx_k0], sems0, add=True)
        g1a.wait()
        g1b.wait()
        g1c.wait()
        paths(ra1, rbj1, rbk1, CH)
        s1 = pltpu.async_copy(ra1, acc.at[idx_k1], sems1, add=True)
        s0.wait()
        s1.wait()
        return _

    lax.fori_loop(0, P_CHUNKS // 2, pair, 0)

    base = w * PPT + P_CHUNKS * CH
    pltpu.sync_copy(pi_hbm.at[pl.ds(base, P_TAIL)], idx_it)
    pltpu.sync_copy(pj_hbm.at[pl.ds(base, P_TAIL)], idx_jt)
    pltpu.sync_copy(pk_hbm.at[pl.ds(base, P_TAIL)], idx_kt)
    cp1 = pltpu.async_copy(a2_hbm.at[idx_it], ra0.at[pl.ds(0, P_TAIL)], semg0)
    cp2 = pltpu.async_copy(dj_hbm.at[idx_jt], rbj0.at[pl.ds(0, P_TAIL)], semg0)
    cp3 = pltpu.async_copy(dk_hbm.at[idx_kt], rbk0.at[pl.ds(0, P_TAIL)], semg0)
    cp1.wait()
    cp2.wait()
    cp3.wait()
    for e in range(P_TAIL):
        _edge_math2(ra0, rbj0, rbk0, e)
    pltpu.sync_copy(ra0.at[pl.ds(0, P_TAIL)], acc.at[idx_kt], add=True)

    plsc.subcore_barrier()

    @pl.when(jnp.logical_and(sid == 0, cid == 0))
    def _():
        pltpu.sync_copy(acc, q0_hbm)

    @pl.when(jnp.logical_and(sid == 0, cid == 1))
    def _():
        pltpu.sync_copy(acc, q1_hbm)


def _pack_weights1(W1, a1_src, a1_dst):
    w3 = W1.reshape(300, H1, C1)
    ws = jnp.einsum('khc,hc->kh', w3, a1_src)
    wd = jnp.einsum('khc,hc->kh', w3, a1_dst)
    z = jnp.zeros((300, 12), jnp.float32)
    return jnp.concatenate([W1, ws, z, wd, z], axis=1)


def _pack_weights2(W2, a2_i, a2_j, a2_k):
    w3 = W2.reshape(F1, H2, C2)
    wi = jnp.einsum('khc,hc->kh', w3, a2_i)
    wj = jnp.einsum('khc,hc->kh', w3, a2_j)
    wk = jnp.einsum('khc,hc->kh', w3, a2_k)
    z = jnp.zeros((F1, 12), jnp.float32)
    return jnp.concatenate([W2, wi, z, wj, z, wk, z], axis=1)


def _den_expand(total_w, heads, width):
    em = jnp.kron(jnp.eye(heads, dtype=jnp.float32),
                  jnp.ones((1, width), jnp.float32))
    top = jnp.zeros((heads * width, heads * width), jnp.float32)
    bot = jnp.zeros((total_w - heads * width - heads, heads * width), jnp.float32)
    return jnp.concatenate([top, em, bot], axis=0)


def kernel(x, edge_index, sec_order_edge_index, W1, a1_src, a1_dst, b1,
           W2, a2_i, a2_j, a2_k, b2):
    src, dst = edge_index[0], edge_index[1]
    pi, pj, pk = (sec_order_edge_index[0], sec_order_edge_index[1],
                  sec_order_edge_index[2])

    we1 = _pack_weights1(W1, a1_src, a1_dst)
    y1 = _matmul(x, we1)
    a1 = y1[:, :A1W]
    d1 = y1[:, A1W:]

    z1 = jnp.zeros((N, A1W), jnp.float32)
    p0, p1 = _make_gat_scatter()(a1, d1, src, dst, z1)

    we2 = _pack_weights2(W2, a2_i, a2_j, a2_k)
    em1 = _den_expand(A1W, H1, C1)
    y2 = _combine1(p0, p1, em1, b1.reshape(1, F1), we2)
    a2 = y2[:, :A2W]
    d2j = y2[:, A2W:A2W + DW]
    d2k = y2[:, A2W + DW:]

    z2 = jnp.zeros((N, A2W), jnp.float32)
    q0, q1 = _make_path_scatter()(a2, d2j, d2k, pi, pj, pk, z2)

    em2 = _den_expand(A2W, H2, C2)
    return _combine2(q0, q1, em2, b2.reshape(1, F2))


# speedup vs baseline: 74.7729x; 1.0047x over previous
import functools

import jax
import jax.numpy as jnp
from jax import lax
from jax.experimental import pallas as pl
from jax.experimental.pallas import tpu as pltpu
from jax.experimental.pallas import tpu_sc as plsc

N = 10000
E = 160000
M = 320000
H1, C1 = 4, 16
H2, C2 = 4, 8
F1 = H1 * C1
F2 = H2 * C2
A1W = F1 + 16
A2W = F2 + 16
DW = 16
RB = 1000
NSC = 2
NTILE = 32
EPT = E // NTILE
PPT = M // NTILE
CH = 128
E_CHUNKS, E_TAIL = EPT // CH, EPT % CH
P_CHUNKS, P_TAIL = PPT // CH, PPT % CH
ZROWS = 624
ZTAIL = N - 16 * ZROWS


def _mm_body(x_ref, w_ref, o_ref):
    o_ref[...] = jnp.dot(x_ref[...], w_ref[...], preferred_element_type=jnp.float32)


def _matmul(x, w):
    n, k = x.shape
    ko, c = w.shape
    return pl.pallas_call(
        _mm_body,
        grid=(n // RB,),
        in_specs=[
            pl.BlockSpec((RB, k), lambda i: (i, 0)),
            pl.BlockSpec((k, c), lambda i: (0, 0)),
        ],
        out_specs=pl.BlockSpec((RB, c), lambda i: (i, 0)),
        out_shape=jax.ShapeDtypeStruct((n, c), jnp.float32),
    )(x, w)


def _combine1_body(p0_ref, p1_ref, em_ref, b1_ref, w2_ref, o_ref):
    t = p0_ref[...] + p1_ref[...]
    den = jnp.dot(t, em_ref[...], preferred_element_type=jnp.float32)
    num = t[:, :F1]
    out1 = jax.nn.relu(num / (den + 1e-16) + b1_ref[...])
    o_ref[...] = jnp.dot(out1, w2_ref[...], preferred_element_type=jnp.float32)


def _combine1(p0, p1, emat, b1r, we2):
    c = we2.shape[1]
    return pl.pallas_call(
        _combine1_body,
        grid=(N // RB,),
        in_specs=[
            pl.BlockSpec((RB, A1W), lambda i: (i, 0)),
            pl.BlockSpec((RB, A1W), lambda i: (i, 0)),
            pl.BlockSpec((A1W, F1), lambda i: (0, 0)),
            pl.BlockSpec((1, F1), lambda i: (0, 0)),
            pl.BlockSpec((F1, c), lambda i: (0, 0)),
        ],
        out_specs=pl.BlockSpec((RB, c), lambda i: (i, 0)),
        out_shape=jax.ShapeDtypeStruct((N, c), jnp.float32),
    )(p0, p1, emat, b1r, we2)


def _combine2_body(p0_ref, p1_ref, em_ref, b2_ref, o_ref):
    t = p0_ref[...] + p1_ref[...]
    den = jnp.dot(t, em_ref[...], preferred_element_type=jnp.float32)
    num = t[:, :F2]
    o_ref[...] = num / (den + 1e-16) + b2_ref[...]


def _combine2(p0, p1, emat, b2r):
    return pl.pallas_call(
        _combine2_body,
        grid=(N // RB,),
        in_specs=[
            pl.BlockSpec((RB, A2W), lambda i: (i, 0)),
            pl.BlockSpec((RB, A2W), lambda i: (i, 0)),
            pl.BlockSpec((A2W, F2), lambda i: (0, 0)),
            pl.BlockSpec((1, F2), lambda i: (0, 0)),
        ],
        out_specs=pl.BlockSpec((RB, F2), lambda i: (i, 0)),
        out_shape=jax.ShapeDtypeStruct((N, F2), jnp.float32),
    )(p0, p1, emat, b2r)


def _splat(vec, idx16):
    return lax.gather(
        vec, idx16.reshape(16, 1),
        lax.GatherDimensionNumbers(
            offset_dims=(), collapsed_slice_dims=(0,), start_index_map=(0,)),
        (1,), mode=lax.GatherScatterMode.PROMISE_IN_BOUNDS)


def _edge_math1(ra, rb, e):
    lanes = lax.iota(jnp.int32, 16)
    mask4 = lanes < 4
    v = ra[e, pl.ds(F1, 16)] + rb[e, :]
    v = jnp.maximum(v, 0.2 * v)
    ex = jnp.where(mask4, jnp.exp(v), 0.0)
    ra[e, pl.ds(F1, 16)] = ex
    for j in range(H1):
        m = _splat(ex, lanes * 0 + j)
        ra[e, pl.ds(C1 * j, 16)] = ra[e, pl.ds(C1 * j, 16)] * m


def _edge_math2(ra, rbj, rbk, e):
    lanes = lax.iota(jnp.int32, 16)
    mask4 = lanes < 4
    v = ra[e, pl.ds(F2, 16)] + rbj[e, :] + rbk[e, :]
    v = jnp.maximum(v, 0.2 * v)
    ex = jnp.where(mask4, jnp.exp(v), 0.0)
    ra[e, pl.ds(F2, 16)] = ex
    half = lanes >> 3
    m0 = _splat(ex, half)
    ra[e, pl.ds(0, 16)] = ra[e, pl.ds(0, 16)] * m0
    m1 = _splat(ex, half + 2)
    ra[e, pl.ds(16, 16)] = ra[e, pl.ds(16, 16)] * m1


@functools.lru_cache(maxsize=None)
def _make_gat_scatter():
    return functools.partial(
        pl.kernel,
        out_type=[jax.ShapeDtypeStruct((N, A1W), jnp.float32),
                  jax.ShapeDtypeStruct((N, A1W), jnp.float32)],
        mesh=plsc.VectorSubcoreMesh(core_axis_name="c", subcore_axis_name="s"),
        compiler_params=pltpu.CompilerParams(use_tc_tiling_on_sc=False),
        scratch_types=[
        pltpu.VMEM_SHARED((N, A1W), jnp.float32),
        pltpu.VMEM((CH, A1W), jnp.float32),
        pltpu.VMEM((CH, A1W), jnp.float32),
        pltpu.VMEM((CH, DW), jnp.float32),
        pltpu.VMEM((CH, DW), jnp.float32),
        pltpu.VMEM((CH,), jnp.int32),
        pltpu.VMEM((CH,), jnp.int32),
        pltpu.VMEM((CH,), jnp.int32),
        pltpu.VMEM((CH,), jnp.int32),
        pltpu.VMEM((E_TAIL,), jnp.int32),
        pltpu.VMEM((E_TAIL,), jnp.int32),
        pltpu.SemaphoreType.DMA,
        pltpu.SemaphoreType.DMA,
        pltpu.SemaphoreType.DMA,
        pltpu.SemaphoreType.DMA,
        ],
    )(_gat_scatter_body)


def _gat_scatter_body(a1_hbm, d1_hbm, src_hbm, dst_hbm, z_hbm, p0_hbm, p1_hbm,
                      acc, ra0, ra1, rb0, rb1, idx_s0, idx_s1, idx_d0, idx_d1,
                      idx_st, idx_dt, semg0, semg1, sems0, sems1):
    cid = lax.axis_index("c")
    sid = lax.axis_index("s")
    w = sid * NSC + cid

    pltpu.sync_copy(z_hbm.at[pl.ds(sid * ZROWS, ZROWS)],
                    acc.at[pl.ds(sid * ZROWS, ZROWS)])

    @pl.when(sid == 0)
    def _():
        pltpu.sync_copy(z_hbm.at[pl.ds(16 * ZROWS, ZTAIL)],
                        acc.at[pl.ds(16 * ZROWS, ZTAIL)])

    plsc.subcore_barrier()

    def edges(ra, rb, n):
        def edge(i, carry):
            for u in range(4):
                _edge_math1(ra, rb, i * 4 + u)
            return carry
        lax.fori_loop(0, n // 4, edge, 0)

    def pair(p, _):
        base_e = w * EPT + (2 * p) * CH
        base_o = base_e + CH
        pltpu.sync_copy(src_hbm.at[pl.ds(base_e, CH)], idx_s0)
        pltpu.sync_copy(dst_hbm.at[pl.ds(base_e, CH)], idx_d0)
        g0a = pltpu.async_copy(a1_hbm.at[idx_s0], ra0, semg0)
        g0b = pltpu.async_copy(d1_hbm.at[idx_d0], rb0, semg0)
        pltpu.sync_copy(src_hbm.at[pl.ds(base_o, CH)], idx_s1)
        pltpu.sync_copy(dst_hbm.at[pl.ds(base_o, CH)], idx_d1)
        g1a = pltpu.async_copy(a1_hbm.at[idx_s1], ra1, semg1)
        g1b = pltpu.async_copy(d1_hbm.at[idx_d1], rb1, semg1)
        g0a.wait()
        g0b.wait()
        edges(ra0, rb0, CH)
        s0 = pltpu.async_copy(ra0, acc.at[idx_d0], sems0, add=True)
        g1a.wait()
        g1b.wait()
        edges(ra1, rb1, CH)
        s1 = pltpu.async_copy(ra1, acc.at[idx_d1], sems1, add=True)
        s0.wait()
        s1.wait()
        return _

    lax.fori_loop(0, E_CHUNKS // 2, pair, 0)

    def chunk(c, _):
        base = w * EPT + c * CH
        pltpu.sync_copy(src_hbm.at[pl.ds(base, CH)], idx_s0)
        pltpu.sync_copy(dst_hbm.at[pl.ds(base, CH)], idx_d0)
        cp1 = pltpu.async_copy(a1_hbm.at[idx_s0], ra0, semg0)
        cp2 = pltpu.async_copy(d1_hbm.at[idx_d0], rb0, semg0)
        cp1.wait()
        cp2.wait()
        edges(ra0, rb0, CH)
        pltpu.sync_copy(ra0, acc.at[idx_d0], add=True)
        return _

    lax.fori_loop(2 * (E_CHUNKS // 2), E_CHUNKS, chunk, 0)

    base = w * EPT + E_CHUNKS * CH
    pltpu.sync_copy(src_hbm.at[pl.ds(base, E_TAIL)], idx_st)
    pltpu.sync_copy(dst_hbm.at[pl.ds(base, E_TAIL)], idx_dt)
    cp1 = pltpu.async_copy(a1_hbm.at[idx_st], ra0.at[pl.ds(0, E_TAIL)], semg0)
    cp2 = pltpu.async_copy(d1_hbm.at[idx_dt], rb0.at[pl.ds(0, E_TAIL)], semg0)
    cp1.wait()
    cp2.wait()
    for e in range(E_TAIL):
        _edge_math1(ra0, rb0, e)
    pltpu.sync_copy(ra0.at[pl.ds(0, E_TAIL)], acc.at[idx_dt], add=True)

    plsc.subcore_barrier()

    @pl.when(jnp.logical_and(sid == 0, cid == 0))
    def _():
        pltpu.sync_copy(acc, p0_hbm)

    @pl.when(jnp.logical_and(sid == 0, cid == 1))
    def _():
        pltpu.sync_copy(acc, p1_hbm)


@functools.lru_cache(maxsize=None)
def _make_path_scatter():
    return functools.partial(
        pl.kernel,
        out_type=[jax.ShapeDtypeStruct((N, A2W), jnp.float32),
                  jax.ShapeDtypeStruct((N, A2W), jnp.float32)],
        mesh=plsc.VectorSubcoreMesh(core_axis_name="c", subcore_axis_name="s"),
        compiler_params=pltpu.CompilerParams(use_tc_tiling_on_sc=False),
        scratch_types=[
        pltpu.VMEM_SHARED((N, A2W), jnp.float32),
        pltpu.VMEM((CH, A2W), jnp.float32),
        pltpu.VMEM((CH, A2W), jnp.float32),
        pltpu.VMEM((CH, DW), jnp.float32),
        pltpu.VMEM((CH, DW), jnp.float32),
        pltpu.VMEM((CH, DW), jnp.float32),
        pltpu.VMEM((CH, DW), jnp.float32),
        pltpu.VMEM((CH,), jnp.int32),
        pltpu.VMEM((CH,), jnp.int32),
        pltpu.VMEM((CH,), jnp.int32),
        pltpu.VMEM((CH,), jnp.int32),
        pltpu.VMEM((CH,), jnp.int32),
        pltpu.VMEM((CH,), jnp.int32),
        pltpu.VMEM((P_TAIL,), jnp.int32),
        pltpu.VMEM((P_TAIL,), jnp.int32),
        pltpu.VMEM((P_TAIL,), jnp.int32),
        pltpu.SemaphoreType.DMA,
        pltpu.SemaphoreType.DMA,
        pltpu.SemaphoreType.DMA,
        pltpu.SemaphoreType.DMA,
        ],
    )(_path_scatter_body)


def _path_scatter_body(a2_hbm, dj_hbm, dk_hbm, pi_hbm, pj_hbm, pk_hbm, z_hbm,
                       q0_hbm, q1_hbm, acc, ra0, ra1, rbj0, rbj1, rbk0, rbk1,
                       idx_i0, idx_i1, idx_j0, idx_j1, idx_k0, idx_k1,
                       idx_it, idx_jt, idx_kt, semg0, semg1, sems0, sems1):
    cid = lax.axis_index("c")
    sid = lax.axis_index("s")
    w = sid * NSC + cid

    pltpu.sync_copy(z_hbm.at[pl.ds(sid * ZROWS, ZROWS)],
                    acc.at[pl.ds(sid * ZROWS, ZROWS)])

    @pl.when(sid == 0)
    def _():
        pltpu.sync_copy(z_hbm.at[pl.ds(16 * ZROWS, ZTAIL)],
                        acc.at[pl.ds(16 * ZROWS, ZTAIL)])

    plsc.subcore_barrier()

    def paths(ra, rbj, rbk, n):
        def path(i, carry):
            for u in range(4):
                _edge_math2(ra, rbj, rbk, i * 4 + u)
            return carry
        lax.fori_loop(0, n // 4, path, 0)

    def pair(p, _):
        base_e = w * PPT + (2 * p) * CH
        base_o = base_e + CH
        pltpu.sync_copy(pi_hbm.at[pl.ds(base_e, CH)], idx_i0)
        pltpu.sync_copy(pj_hbm.at[pl.ds(base_e, CH)], idx_j0)
        pltpu.sync_copy(pk_hbm.at[pl.ds(base_e, CH)], idx_k0)
        g0a = pltpu.async_copy(a2_hbm.at[idx_i0], ra0, semg0)
        g0b = pltpu.async_copy(dj_hbm.at[idx_j0], rbj0, semg0)
        g0c = pltpu.async_copy(dk_hbm.at[idx_k0], rbk0, semg0)
        pltpu.sync_copy(pi_hbm.at[pl.ds(base_o, CH)], idx_i1)
        pltpu.sync_copy(pj_hbm.at[pl.ds(base_o, CH)], idx_j1)
        pltpu.sync_copy(pk_hbm.at[pl.ds(base_o, CH)], idx_k1)
        g1a = pltpu.async_copy(a2_hbm.at[idx_i1], ra1, semg1)
        g1b = pltpu.async_copy(dj_hbm.at[idx_j1], rbj1, semg1)
        g1c = pltpu.async_copy(dk_hbm.at[idx_k1], rbk1, semg1)
        g0a.wait()
        g0b.wait()
        g0c.wait()
        paths(ra0, rbj0, rbk0, CH)
        s0 = pltpu.async_copy(ra0, acc.at[idx_k0], sems0, add=True)
        g1a.wait()
        g1b.wait()
        g1c.wait()
        paths(ra1, rbj1, rbk1, CH)
        s1 = pltpu.async_copy(ra1, acc.at[idx_k1], sems1, add=True)
        s0.wait()
        s1.wait()
        return _

    lax.fori_loop(0, P_CHUNKS // 2, pair, 0)

    base = w * PPT + P_CHUNKS * CH
    pltpu.sync_copy(pi_hbm.at[pl.ds(base, P_TAIL)], idx_it)
    pltpu.sync_copy(pj_hbm.at[pl.ds(base, P_TAIL)], idx_jt)
    pltpu.sync_copy(pk_hbm.at[pl.ds(base, P_TAIL)], idx_kt)
    cp1 = pltpu.async_copy(a2_hbm.at[idx_it], ra0.at[pl.ds(0, P_TAIL)], semg0)
    cp2 = pltpu.async_copy(dj_hbm.at[idx_jt], rbj0.at[pl.ds(0, P_TAIL)], semg0)
    cp3 = pltpu.async_copy(dk_hbm.at[idx_kt], rbk0.at[pl.ds(0, P_TAIL)], semg0)
    cp1.wait()
    cp2.wait()
    cp3.wait()
    for e in range(P_TAIL):
        _edge_math2(ra0, rbj0, rbk0, e)
    pltpu.sync_copy(ra0.at[pl.ds(0, P_TAIL)], acc.at[idx_kt], add=True)

    plsc.subcore_barrier()

    @pl.when(jnp.logical_and(sid == 0, cid == 0))
    def _():
        pltpu.sync_copy(acc, q0_hbm)

    @pl.when(jnp.logical_and(sid == 0, cid == 1))
    def _():
        pltpu.sync_copy(acc, q1_hbm)


def _pack_weights1(W1, a1_src, a1_dst):
    w3 = W1.reshape(300, H1, C1)
    ws = jnp.einsum('khc,hc->kh', w3, a1_src)
    wd = jnp.einsum('khc,hc->kh', w3, a1_dst)
    z = jnp.zeros((300, 12), jnp.float32)
    return jnp.concatenate([W1, ws, z, wd, z], axis=1)


def _pack_weights2(W2, a2_i, a2_j, a2_k):
    w3 = W2.reshape(F1, H2, C2)
    wi = jnp.einsum('khc,hc->kh', w3, a2_i)
    wj = jnp.einsum('khc,hc->kh', w3, a2_j)
    wk = jnp.einsum('khc,hc->kh', w3, a2_k)
    z = jnp.zeros((F1, 12), jnp.float32)
    return jnp.concatenate([W2, wi, z, wj, z, wk, z], axis=1)


def _den_expand(total_w, heads, width):
    em = jnp.kron(jnp.eye(heads, dtype=jnp.float32),
                  jnp.ones((1, width), jnp.float32))
    top = jnp.zeros((heads * width, heads * width), jnp.float32)
    bot = jnp.zeros((total_w - heads * width - heads, heads * width), jnp.float32)
    return jnp.concatenate([top, em, bot], axis=0)


def kernel(x, edge_index, sec_order_edge_index, W1, a1_src, a1_dst, b1,
           W2, a2_i, a2_j, a2_k, b2):
    src, dst = edge_index[0], edge_index[1]
    pi, pj, pk = (sec_order_edge_index[0], sec_order_edge_index[1],
                  sec_order_edge_index[2])

    we1 = _pack_weights1(W1, a1_src, a1_dst)
    y1 = _matmul(x, we1)
    a1 = y1[:, :A1W]
    d1 = y1[:, A1W:]

    z1 = jnp.zeros((N, A1W), jnp.float32)
    p0, p1 = _make_gat_scatter()(a1, d1, src, dst, z1)

    we2 = _pack_weights2(W2, a2_i, a2_j, a2_k)
    em1 = _den_expand(A1W, H1, C1)
    y2 = _combine1(p0, p1, em1, b1.reshape(1, F1), we2)
    a2 = y2[:, :A2W]
    d2j = y2[:, A2W:A2W + DW]
    d2k = y2[:, A2W + DW:]

    z2 = jnp.zeros((N, A2W), jnp.float32)
    q0, q1 = _make_path_scatter()(a2, d2j, d2k, pi, pj, pk, z2)

    em2 = _den_expand(A2W, H2, C2)
    return _combine2(q0, q1, em2, b2.reshape(1, F2))


# speedup vs baseline: 84.7348x; 1.1332x over previous
import functools

import jax
import jax.numpy as jnp
from jax import lax
from jax.experimental import pallas as pl
from jax.experimental.pallas import tpu as pltpu
from jax.experimental.pallas import tpu_sc as plsc

N = 10000
E = 160000
M = 320000
H1, C1 = 4, 16
H2, C2 = 4, 8
F1 = H1 * C1
F2 = H2 * C2
A1W = F1 + 16
A2W = F2 + 16
DW = 16
RB = 1000
NSC = 2
NTILE = 32
EPT = E // NTILE
PPT = M // NTILE
CH = 128
E_CHUNKS, E_TAIL = EPT // CH, EPT % CH
P_CHUNKS, P_TAIL = PPT // CH, PPT % CH
ZROWS = 624
ZTAIL = N - 16 * ZROWS


def _mm_body(x_ref, w_ref, o_ref):
    o_ref[...] = jnp.dot(x_ref[...], w_ref[...], preferred_element_type=jnp.float32)


def _matmul(x, w):
    n, k = x.shape
    ko, c = w.shape
    return pl.pallas_call(
        _mm_body,
        grid=(n // RB,),
        in_specs=[
            pl.BlockSpec((RB, k), lambda i: (i, 0)),
            pl.BlockSpec((k, c), lambda i: (0, 0)),
        ],
        out_specs=pl.BlockSpec((RB, c), lambda i: (i, 0)),
        out_shape=jax.ShapeDtypeStruct((n, c), jnp.float32),
    )(x, w)


def _combine1_body(p0_ref, p1_ref, em_ref, b1_ref, w2_ref, o_ref):
    t = p0_ref[...] + p1_ref[...]
    den = jnp.dot(t, em_ref[...], preferred_element_type=jnp.float32)
    num = t[:, :F1]
    out1 = jax.nn.relu(num / (den + 1e-16) + b1_ref[...])
    o_ref[...] = jnp.dot(out1, w2_ref[...], preferred_element_type=jnp.float32)


def _combine1(p0, p1, emat, b1r, we2):
    c = we2.shape[1]
    return pl.pallas_call(
        _combine1_body,
        grid=(N // RB,),
        in_specs=[
            pl.BlockSpec((RB, A1W), lambda i: (i, 0)),
            pl.BlockSpec((RB, A1W), lambda i: (i, 0)),
            pl.BlockSpec((A1W, F1), lambda i: (0, 0)),
            pl.BlockSpec((1, F1), lambda i: (0, 0)),
            pl.BlockSpec((F1, c), lambda i: (0, 0)),
        ],
        out_specs=pl.BlockSpec((RB, c), lambda i: (i, 0)),
        out_shape=jax.ShapeDtypeStruct((N, c), jnp.float32),
    )(p0, p1, emat, b1r, we2)


def _combine2_body(p0_ref, p1_ref, em_ref, b2_ref, o_ref):
    t = p0_ref[...] + p1_ref[...]
    den = jnp.dot(t, em_ref[...], preferred_element_type=jnp.float32)
    num = t[:, :F2]
    o_ref[...] = num / (den + 1e-16) + b2_ref[...]


def _combine2(p0, p1, emat, b2r):
    return pl.pallas_call(
        _combine2_body,
        grid=(N // RB,),
        in_specs=[
            pl.BlockSpec((RB, A2W), lambda i: (i, 0)),
            pl.BlockSpec((RB, A2W), lambda i: (i, 0)),
            pl.BlockSpec((A2W, F2), lambda i: (0, 0)),
            pl.BlockSpec((1, F2), lambda i: (0, 0)),
        ],
        out_specs=pl.BlockSpec((RB, F2), lambda i: (i, 0)),
        out_shape=jax.ShapeDtypeStruct((N, F2), jnp.float32),
    )(p0, p1, emat, b2r)


def _splat(vec, idx16):
    return lax.gather(
        vec, idx16.reshape(16, 1),
        lax.GatherDimensionNumbers(
            offset_dims=(), collapsed_slice_dims=(0,), start_index_map=(0,)),
        (1,), mode=lax.GatherScatterMode.PROMISE_IN_BOUNDS)


def _edge_math1(ra, rb, e):
    lanes = lax.iota(jnp.int32, 16)
    mask4 = lanes < 4
    v = ra[e, pl.ds(F1, 16)] + rb[e, :]
    v = jnp.maximum(v, 0.2 * v)
    ex = jnp.where(mask4, jnp.exp(v), 0.0)
    ra[e, pl.ds(F1, 16)] = ex
    for j in range(H1):
        m = _splat(ex, lanes * 0 + j)
        ra[e, pl.ds(C1 * j, 16)] = ra[e, pl.ds(C1 * j, 16)] * m


def _edge_math2(ra, rbj, rbk, e):
    lanes = lax.iota(jnp.int32, 16)
    mask4 = lanes < 4
    v = ra[e, pl.ds(F2, 16)] + rbj[e, :] + rbk[e, :]
    v = jnp.maximum(v, 0.2 * v)
    ex = jnp.where(mask4, jnp.exp(v), 0.0)
    ra[e, pl.ds(F2, 16)] = ex
    half = lanes >> 3
    m0 = _splat(ex, half)
    ra[e, pl.ds(0, 16)] = ra[e, pl.ds(0, 16)] * m0
    m1 = _splat(ex, half + 2)
    ra[e, pl.ds(16, 16)] = ra[e, pl.ds(16, 16)] * m1


@functools.lru_cache(maxsize=None)
def _make_gat_scatter():
    return functools.partial(
        pl.kernel,
        out_type=[jax.ShapeDtypeStruct((N, A1W), jnp.float32),
                  jax.ShapeDtypeStruct((N, A1W), jnp.float32)],
        mesh=plsc.VectorSubcoreMesh(core_axis_name="c", subcore_axis_name="s"),
        compiler_params=pltpu.CompilerParams(use_tc_tiling_on_sc=False),
        scratch_types=[
        pltpu.VMEM_SHARED((N, A1W), jnp.float32),
        pltpu.VMEM((CH, A1W), jnp.float32),
        pltpu.VMEM((CH, A1W), jnp.float32),
        pltpu.VMEM((CH, DW), jnp.float32),
        pltpu.VMEM((CH, DW), jnp.float32),
        pltpu.VMEM((EPT,), jnp.int32),
        pltpu.VMEM((EPT,), jnp.int32),
        pltpu.VMEM((CH,), jnp.int32),
        pltpu.VMEM((CH,), jnp.int32),
        pltpu.VMEM((E_TAIL,), jnp.int32),
        pltpu.SemaphoreType.DMA,
        pltpu.SemaphoreType.DMA,
        pltpu.SemaphoreType.DMA,
        pltpu.SemaphoreType.DMA,
        pltpu.SemaphoreType.DMA,
        pltpu.SemaphoreType.DMA,
        ],
    )(_gat_scatter_body)


def _gat_scatter_body(a1_hbm, d1_hbm, src_hbm, dst_hbm, z_hbm, p0_hbm, p1_hbm,
                      acc, ra0, ra1, rb0, rb1, sidx, didx, idx_d0, idx_d1,
                      idx_dt, semg0, semg1, sems0, sems1, semi0, semi1):
    cid = lax.axis_index("c")
    sid = lax.axis_index("s")
    w = sid * NSC + cid

    pltpu.sync_copy(src_hbm.at[pl.ds(w * EPT, EPT)], sidx)
    pltpu.sync_copy(dst_hbm.at[pl.ds(w * EPT, EPT)], didx)

    pltpu.sync_copy(z_hbm.at[pl.ds(sid * ZROWS, ZROWS)],
                    acc.at[pl.ds(sid * ZROWS, ZROWS)])

    @pl.when(sid == 0)
    def _():
        pltpu.sync_copy(z_hbm.at[pl.ds(16 * ZROWS, ZTAIL)],
                        acc.at[pl.ds(16 * ZROWS, ZTAIL)])

    plsc.subcore_barrier()

    def edges(ra, rb, n):
        def edge(i, carry):
            for u in range(4):
                _edge_math1(ra, rb, i * 4 + u)
            return carry
        lax.fori_loop(0, n // 4, edge, 0)

    def pair(p, _):
        o_e = (2 * p) * CH
        o_o = o_e + CH
        g0a = pltpu.async_copy(a1_hbm.at[sidx.at[pl.ds(o_e, CH)]], ra0, semg0)
        g0b = pltpu.async_copy(d1_hbm.at[didx.at[pl.ds(o_e, CH)]], rb0, semg0)
        g1a = pltpu.async_copy(a1_hbm.at[sidx.at[pl.ds(o_o, CH)]], ra1, semg1)
        g1b = pltpu.async_copy(d1_hbm.at[didx.at[pl.ds(o_o, CH)]], rb1, semg1)
        i0 = pltpu.async_copy(dst_hbm.at[pl.ds(w * EPT + o_e, CH)], idx_d0,
                              semi0)
        i1 = pltpu.async_copy(dst_hbm.at[pl.ds(w * EPT + o_o, CH)], idx_d1,
                              semi1)
        g0a.wait()
        g0b.wait()
        edges(ra0, rb0, CH)
        i0.wait()
        s0 = pltpu.async_copy(ra0, acc.at[idx_d0], sems0, add=True)
        g1a.wait()
        g1b.wait()
        edges(ra1, rb1, CH)
        i1.wait()
        s1 = pltpu.async_copy(ra1, acc.at[idx_d1], sems1, add=True)
        s0.wait()
        s1.wait()
        return _

    lax.fori_loop(0, E_CHUNKS // 2, pair, 0)

    def chunk(c, _):
        o = c * CH
        cp1 = pltpu.async_copy(a1_hbm.at[sidx.at[pl.ds(o, CH)]], ra0, semg0)
        cp2 = pltpu.async_copy(d1_hbm.at[didx.at[pl.ds(o, CH)]], rb0, semg0)
        i0 = pltpu.async_copy(dst_hbm.at[pl.ds(w * EPT + o, CH)], idx_d0,
                              semi0)
        cp1.wait()
        cp2.wait()
        edges(ra0, rb0, CH)
        i0.wait()
        pltpu.sync_copy(ra0, acc.at[idx_d0], add=True)
        return _

    lax.fori_loop(2 * (E_CHUNKS // 2), E_CHUNKS, chunk, 0)

    o = E_CHUNKS * CH
    cp1 = pltpu.async_copy(a1_hbm.at[sidx.at[pl.ds(o, E_TAIL)]],
                           ra0.at[pl.ds(0, E_TAIL)], semg0)
    cp2 = pltpu.async_copy(d1_hbm.at[didx.at[pl.ds(o, E_TAIL)]],
                           rb0.at[pl.ds(0, E_TAIL)], semg0)
    it = pltpu.async_copy(dst_hbm.at[pl.ds(w * EPT + o, E_TAIL)], idx_dt,
                          semi0)
    it.wait()
    cp1.wait()
    cp2.wait()
    for e in range(E_TAIL):
        _edge_math1(ra0, rb0, e)
    pltpu.sync_copy(ra0.at[pl.ds(0, E_TAIL)], acc.at[idx_dt], add=True)

    plsc.subcore_barrier()

    @pl.when(jnp.logical_and(sid == 0, cid == 0))
    def _():
        pltpu.sync_copy(acc, p0_hbm)

    @pl.when(jnp.logical_and(sid == 0, cid == 1))
    def _():
        pltpu.sync_copy(acc, p1_hbm)


@functools.lru_cache(maxsize=None)
def _make_path_scatter():
    return functools.partial(
        pl.kernel,
        out_type=[jax.ShapeDtypeStruct((N, A2W), jnp.float32),
                  jax.ShapeDtypeStruct((N, A2W), jnp.float32)],
        mesh=plsc.VectorSubcoreMesh(core_axis_name="c", subcore_axis_name="s"),
        compiler_params=pltpu.CompilerParams(use_tc_tiling_on_sc=False),
        scratch_types=[
        pltpu.VMEM_SHARED((N, A2W), jnp.float32),
        pltpu.VMEM((CH, A2W), jnp.float32),
        pltpu.VMEM((CH, A2W), jnp.float32),
        pltpu.VMEM((CH, DW), jnp.float32),
        pltpu.VMEM((CH, DW), jnp.float32),
        pltpu.VMEM((CH, DW), jnp.float32),
        pltpu.VMEM((CH, DW), jnp.float32),
        pltpu.VMEM((PPT,), jnp.int32),
        pltpu.VMEM((PPT,), jnp.int32),
        pltpu.VMEM((PPT,), jnp.int32),
        pltpu.VMEM((CH,), jnp.int32),
        pltpu.VMEM((CH,), jnp.int32),
        pltpu.VMEM((P_TAIL,), jnp.int32),
        pltpu.SemaphoreType.DMA,
        pltpu.SemaphoreType.DMA,
        pltpu.SemaphoreType.DMA,
        pltpu.SemaphoreType.DMA,
        pltpu.SemaphoreType.DMA,
        pltpu.SemaphoreType.DMA,
        ],
    )(_path_scatter_body)


def _path_scatter_body(a2_hbm, dj_hbm, dk_hbm, pi_hbm, pj_hbm, pk_hbm, z_hbm,
                       q0_hbm, q1_hbm, acc, ra0, ra1, rbj0, rbj1, rbk0, rbk1,
                       iidx, jidx, kidx, idx_k0, idx_k1,
                       idx_kt, semg0, semg1, sems0, sems1, semi0, semi1):
    cid = lax.axis_index("c")
    sid = lax.axis_index("s")
    w = sid * NSC + cid

    pltpu.sync_copy(pi_hbm.at[pl.ds(w * PPT, PPT)], iidx)
    pltpu.sync_copy(pj_hbm.at[pl.ds(w * PPT, PPT)], jidx)
    pltpu.sync_copy(pk_hbm.at[pl.ds(w * PPT, PPT)], kidx)

    pltpu.sync_copy(z_hbm.at[pl.ds(sid * ZROWS, ZROWS)],
                    acc.at[pl.ds(sid * ZROWS, ZROWS)])

    @pl.when(sid == 0)
    def _():
        pltpu.sync_copy(z_hbm.at[pl.ds(16 * ZROWS, ZTAIL)],
                        acc.at[pl.ds(16 * ZROWS, ZTAIL)])

    plsc.subcore_barrier()

    def paths(ra, rbj, rbk, n):
        def path(i, carry):
            for u in range(4):
                _edge_math2(ra, rbj, rbk, i * 4 + u)
            return carry
        lax.fori_loop(0, n // 4, path, 0)

    def pair(p, _):
        o_e = (2 * p) * CH
        o_o = o_e + CH
        g0a = pltpu.async_copy(a2_hbm.at[iidx.at[pl.ds(o_e, CH)]], ra0, semg0)
        g0b = pltpu.async_copy(dj_hbm.at[jidx.at[pl.ds(o_e, CH)]], rbj0, semg0)
        g0c = pltpu.async_copy(dk_hbm.at[kidx.at[pl.ds(o_e, CH)]], rbk0, semg0)
        g1a = pltpu.async_copy(a2_hbm.at[iidx.at[pl.ds(o_o, CH)]], ra1, semg1)
        g1b = pltpu.async_copy(dj_hbm.at[jidx.at[pl.ds(o_o, CH)]], rbj1, semg1)
        g1c = pltpu.async_copy(dk_hbm.at[kidx.at[pl.ds(o_o, CH)]], rbk1, semg1)
        i0 = pltpu.async_copy(pk_hbm.at[pl.ds(w * PPT + o_e, CH)], idx_k0,
                              semi0)
        i1 = pltpu.async_copy(pk_hbm.at[pl.ds(w * PPT + o_o, CH)], idx_k1,
                              semi1)
        g0a.wait()
        g0b.wait()
        g0c.wait()
        paths(ra0, rbj0, rbk0, CH)
        i0.wait()
        s0 = pltpu.async_copy(ra0, acc.at[idx_k0], sems0, add=True)
        g1a.wait()
        g1b.wait()
        g1c.wait()
        paths(ra1, rbj1, rbk1, CH)
        i1.wait()
        s1 = pltpu.async_copy(ra1, acc.at[idx_k1], sems1, add=True)
        s0.wait()
        s1.wait()
        return _

    lax.fori_loop(0, P_CHUNKS // 2, pair, 0)

    o = P_CHUNKS * CH
    cp1 = pltpu.async_copy(a2_hbm.at[iidx.at[pl.ds(o, P_TAIL)]],
                           ra0.at[pl.ds(0, P_TAIL)], semg0)
    cp2 = pltpu.async_copy(dj_hbm.at[jidx.at[pl.ds(o, P_TAIL)]],
                           rbj0.at[pl.ds(0, P_TAIL)], semg0)
    cp3 = pltpu.async_copy(dk_hbm.at[kidx.at[pl.ds(o, P_TAIL)]],
                           rbk0.at[pl.ds(0, P_TAIL)], semg0)
    it = pltpu.async_copy(pk_hbm.at[pl.ds(w * PPT + o, P_TAIL)], idx_kt,
                          semi0)
    it.wait()
    cp1.wait()
    cp2.wait()
    cp3.wait()
    for e in range(P_TAIL):
        _edge_math2(ra0, rbj0, rbk0, e)
    pltpu.sync_copy(ra0.at[pl.ds(0, P_TAIL)], acc.at[idx_kt], add=True)

    plsc.subcore_barrier()

    @pl.when(jnp.logical_and(sid == 0, cid == 0))
    def _():
        pltpu.sync_copy(acc, q0_hbm)

    @pl.when(jnp.logical_and(sid == 0, cid == 1))
    def _():
        pltpu.sync_copy(acc, q1_hbm)


def _pack_weights1(W1, a1_src, a1_dst):
    w3 = W1.reshape(300, H1, C1)
    ws = jnp.einsum('khc,hc->kh', w3, a1_src)
    wd = jnp.einsum('khc,hc->kh', w3, a1_dst)
    z = jnp.zeros((300, 12), jnp.float32)
    return jnp.concatenate([W1, ws, z, wd, z], axis=1)


def _pack_weights2(W2, a2_i, a2_j, a2_k):
    w3 = W2.reshape(F1, H2, C2)
    wi = jnp.einsum('khc,hc->kh', w3, a2_i)
    wj = jnp.einsum('khc,hc->kh', w3, a2_j)
    wk = jnp.einsum('khc,hc->kh', w3, a2_k)
    z = jnp.zeros((F1, 12), jnp.float32)
    return jnp.concatenate([W2, wi, z, wj, z, wk, z], axis=1)


def _den_expand(total_w, heads, width):
    em = jnp.kron(jnp.eye(heads, dtype=jnp.float32),
                  jnp.ones((1, width), jnp.float32))
    top = jnp.zeros((heads * width, heads * width), jnp.float32)
    bot = jnp.zeros((total_w - heads * width - heads, heads * width), jnp.float32)
    return jnp.concatenate([top, em, bot], axis=0)


def kernel(x, edge_index, sec_order_edge_index, W1, a1_src, a1_dst, b1,
           W2, a2_i, a2_j, a2_k, b2):
    src, dst = edge_index[0], edge_index[1]
    pi, pj, pk = (sec_order_edge_index[0], sec_order_edge_index[1],
                  sec_order_edge_index[2])

    we1 = _pack_weights1(W1, a1_src, a1_dst)
    y1 = _matmul(x, we1)
    a1 = y1[:, :A1W]
    d1 = y1[:, A1W:]

    z1 = jnp.zeros((N, A1W), jnp.float32)
    p0, p1 = _make_gat_scatter()(a1, d1, src, dst, z1)

    we2 = _pack_weights2(W2, a2_i, a2_j, a2_k)
    em1 = _den_expand(A1W, H1, C1)
    y2 = _combine1(p0, p1, em1, b1.reshape(1, F1), we2)
    a2 = y2[:, :A2W]
    d2j = y2[:, A2W:A2W + DW]
    d2k = y2[:, A2W + DW:]

    z2 = jnp.zeros((N, A2W), jnp.float32)
    q0, q1 = _make_path_scatter()(a2, d2j, d2k, pi, pj, pk, z2)

    em2 = _den_expand(A2W, H2, C2)
    return _combine2(q0, q1, em2, b2.reshape(1, F2))
